# baseline (speedup 1.0000x reference)
import functools

import jax
import jax.numpy as jnp
from jax import lax
from jax.experimental import pallas as pl
from jax.experimental.pallas import tpu as pltpu
from jax.experimental.pallas import tpu_sc as plsc

N_NODES = 10000
N_EDGES = 160000
T = 256
PAIRS = 2 * T
D = 128
HID = 256
N_PAD = 10240
RP_PAD = 10016
COLS_PAD = N_EDGES + 64
W = 64


def _sc_build(nodes, cols, row_ptr, x):
  info = plsc.get_sparse_core_info()
  NC, NS = info.num_cores, info.num_subcores
  NW = NC * NS
  PPW = PAIRS // NW
  mesh = plsc.VectorSubcoreMesh(core_axis_name="c", subcore_axis_name="s")

  @functools.partial(
      pl.kernel, mesh=mesh,
      compiler_params=pltpu.CompilerParams(needs_layout_passes=False),
      out_type=[
          jax.ShapeDtypeStruct((PAIRS, N_PAD), jnp.float32),
          jax.ShapeDtypeStruct((PAIRS, N_PAD), jnp.float32),
          jax.ShapeDtypeStruct((PAIRS, D), jnp.float32),
      ],
      scratch_types=[
          pltpu.VMEM((PPW + 16,), jnp.int32),
          pltpu.VMEM((RP_PAD,), jnp.int32),
          pltpu.VMEM((N_PAD,), jnp.float32),
          pltpu.VMEM((N_PAD,), jnp.float32),
          pltpu.VMEM((W + 16,), jnp.int32),
          pltpu.VMEM((W,), jnp.int32),
          pltpu.VMEM((PPW, D), jnp.float32),
          pltpu.VMEM_SHARED((COLS_PAD,), jnp.int32),
          pltpu.SemaphoreType.DMA,
      ],
  )
  def k(nodes_h, cols_h, rp_h, x_h, m1_h, m2_h, xg_h,
        nodes_v, rp_v, m1row, m2row, c1, c2, xrows, cols_s, sem):
    cid = lax.axis_index("c")
    sid = lax.axis_index("s")
    wid = sid * NC + cid
    base_pair = wid * PPW

    @pl.when(sid == 0)
    def _():
      pltpu.sync_copy(cols_h, cols_s)
    plsc.subcore_barrier()

    pltpu.sync_copy(rp_h, rp_v)
    pltpu.sync_copy(nodes_h.at[pl.ds(base_pair, PPW)],
                    nodes_v.at[pl.ds(0, PPW)])

    idxv = nodes_v[pl.ds(0, 16)]
    pltpu.async_copy(x_h.at[idxv], xrows, sem).wait()
    pltpu.sync_copy(xrows, xg_h.at[pl.ds(base_pair, PPW)])

    lanes = lax.iota(jnp.int32, 16)
    ones = jnp.ones((16,), jnp.float32)

    def sload(ref, i):
      return ref[pl.ds(i, 16)][0]

    def scatter_window(dst_row, chunk, s, b, e):
      for g in range(W // 16):
        col = chunk[pl.ds(g * 16, 16)]
        gl = g * 16 + lanes
        valid = (gl >= (s - b)) & (gl < (e - b))
        plsc.store_scatter(dst_row, [col], ones, mask=valid)

    def fill_one_hop(dst_row, n):
      s = sload(rp_v, n)
      e = sload(rp_v, n + 1)
      b0 = (s // 8) * 8
      nwin = (e - b0 + (W - 1)) // W

      @pl.loop(0, nwin)
      def _(w):
        b = b0 + w * W
        pltpu.sync_copy(cols_s.at[pl.ds(b, W)], c1.at[pl.ds(0, W)])
        scatter_window(dst_row, c1, s, b, e)

    def fill_two_hop(dst_row, n):
      s = sload(rp_v, n)
      e = sload(rp_v, n + 1)
      b0 = (s // 8) * 8
      nwin = (e - b0 + (W - 1)) // W

      @pl.loop(0, nwin)
      def _(w):
        b = b0 + w * W
        pltpu.sync_copy(cols_s.at[pl.ds(b, W)], c1.at[pl.ds(0, W)])
        jlo = jnp.maximum(s - b, 0)
        jhi = jnp.minimum(e - b, W)

        @pl.loop(jlo, jhi)
        def _(j):
          kk = sload(c1, j)
          s2 = sload(rp_v, kk)
          e2 = sload(rp_v, kk + 1)
          b20 = (s2 // 8) * 8
          nw2 = (e2 - b20 + (W - 1)) // W

          @pl.loop(0, nw2)
          def _(w2):
            b2 = b20 + w2 * W
            pltpu.sync_copy(cols_s.at[pl.ds(b2, W)], c2)
            scatter_window(dst_row, c2, s2, b2, e2)

    def pair_body(p, carry):
      n = sload(nodes_v, p)

      def zbody(i, carry2):
        m1row[pl.ds(i * 16, 16)] = jnp.zeros((16,), jnp.float32)
        m2row[pl.ds(i * 16, 16)] = jnp.zeros((16,), jnp.float32)
        return carry2

      lax.fori_loop(0, N_PAD // 16, zbody, 0)
      fill_one_hop(m1row, n)
      fill_two_hop(m2row, n)
      pg = base_pair + p
      pltpu.sync_copy(m1row, m1_h.at[pg])
      pltpu.sync_copy(m2row, m2_h.at[pg])
      return carry

    lax.fori_loop(0, PPW, pair_body, 0)

  return k(nodes, cols, row_ptr, x)


TN = 1024
GSTEPS = N_PAD // TN


def _tc_body(m1_ref, m2_ref, xb_ref, xg_ref,
             w11, b11, w12, b12, w13, b13,
             w21, b21, w22, b22, w23, b23,
             w41, b41, w42, b42, w43, b43,
             wi1, bi1, wi2, bi2,
             wl1, bl1, wl2, bl2,
             alpha, beta,
             out_ref, a1, a2, a3, a4):
  i = pl.program_id(0)

  @pl.when(i == 0)
  def _():
    a1[...] = jnp.zeros_like(a1)
    a2[...] = jnp.zeros_like(a2)
    a3[...] = jnp.zeros_like(a3)
    a4[...] = jnp.zeros_like(a4)

  m1 = m1_ref[...]
  m2 = m2_ref[...]
  xb = xb_ref[...]
  m1s, m1d = m1[:T, :], m1[T:, :]
  m2s, m2d = m2[:T, :], m2[T:, :]
  dot = functools.partial(jnp.dot, preferred_element_type=jnp.float32)
  a1[...] += dot(m1s * m1d, xb)
  a2[...] += dot(m1s * m2d, xb)
  a3[...] += dot(m2s * m1d, xb)
  a4[...] += dot(m2s * m2d, xb)

  @pl.when(i == GSTEPS - 1)
  def _():
    def mlp3(h, wa, ba, wb, bb, wc, bc):
      h = jnp.maximum(dot(h, wa[...]) + ba[...], 0.0)
      h = jnp.maximum(dot(h, wb[...]) + bb[...], 0.0)
      return dot(h, wc[...]) + bc[...]

    xcn1 = mlp3(a1[...], w11, b11, w12, b12, w13, b13)
    xcn2 = mlp3(a2[...], w21, b21, w22, b22, w23, b23)
    xcn3 = mlp3(a3[...], w21, b21, w22, b22, w23, b23)
    xcn4 = mlp3(a4[...], w41, b41, w42, b42, w43, b43)

    xg = xg_ref[...]
    xij_in = xg[:T, :] * xg[T:, :]
    hij = jnp.maximum(dot(xij_in, wi1[...]) + bi1[...], 0.0)
    xij = dot(hij, wi2[...]) + bi2[...]

    sa = jax.nn.sigmoid(alpha[...])
    al0 = sa[0, 0]
    al1 = al0 * sa[0, 1]
    al2 = al1 * sa[0, 2]
    h = al0 * xcn1 + al1 * xcn2 * xcn3 + al2 * xcn4 + beta[0, 0] * xij
    hl = jnp.maximum(dot(h, wl1[...]) + bl1[...], 0.0)
    out_ref[...] = dot(hl, wl2[...]) + bl2[...]


def _tc_combine(m1, m2, x_pad, xg, flat_w):
  full = lambda arr: pl.BlockSpec(arr.shape, lambda i: (0,) * arr.ndim)
  in_specs = [
      pl.BlockSpec((PAIRS, TN), lambda i: (0, i)),
      pl.BlockSpec((PAIRS, TN), lambda i: (0, i)),
      pl.BlockSpec((TN, D), lambda i: (i, 0)),
      full(xg),
  ] + [full(a) for a in flat_w]
  return pl.pallas_call(
      _tc_body,
      grid=(GSTEPS,),
      in_specs=in_specs,
      out_specs=pl.BlockSpec((T, D), lambda i: (0, 0)),
      out_shape=jax.ShapeDtypeStruct((T, D), jnp.float32),
      scratch_shapes=[pltpu.VMEM((T, D), jnp.float32)] * 4,
  )(m1, m2, x_pad, xg, *flat_w)


def kernel(x, adj, tar_ei, params):
  adj0 = adj[0].astype(jnp.int32)
  adj1 = adj[1].astype(jnp.int32)

  order = jnp.arange(N_EDGES, dtype=jnp.int32)
  rows_sorted = adj0[order]
  cols_sorted = adj1[order]
  row_ptr = jnp.searchsorted(
      rows_sorted, jnp.arange(N_NODES + 1, dtype=jnp.int32)).astype(jnp.int32)
  row_ptr = jnp.pad(row_ptr, (0, RP_PAD - (N_NODES + 1)),
                    constant_values=N_EDGES)
  cols_pad = jnp.pad(cols_sorted, (0, COLS_PAD - N_EDGES))
  nodes = jnp.concatenate([tar_ei[0], tar_ei[1]]).astype(jnp.int32)

  m1, m2, xg = _sc_build(nodes, cols_pad, row_ptr, x)

  x_pad = jnp.pad(x, ((0, N_PAD - N_NODES), (0, 0)))

  p = params
  row = lambda b: b.reshape(1, -1)
  wl2 = jnp.pad(p["lin"][1]["W"], ((0, 0), (0, D - 1)))
  bl2 = jnp.pad(row(p["lin"][1]["b"]), ((0, 0), (0, D - 1)))
  flat_w = [
      p["xcn1"][0]["W"], row(p["xcn1"][0]["b"]),
      p["xcn1"][1]["W"], row(p["xcn1"][1]["b"]),
      p["xcn1"][2]["W"], row(p["xcn1"][2]["b"]),
      p["xcn2"][0]["W"], row(p["xcn2"][0]["b"]),
      p["xcn2"][1]["W"], row(p["xcn2"][1]["b"]),
      p["xcn2"][2]["W"], row(p["xcn2"][2]["b"]),
      p["xcn4"][0]["W"], row(p["xcn4"][0]["b"]),
      p["xcn4"][1]["W"], row(p["xcn4"][1]["b"]),
      p["xcn4"][2]["W"], row(p["xcn4"][2]["b"]),
      p["xij"][0]["W"], row(p["xij"][0]["b"]),
      p["xij"][1]["W"], row(p["xij"][1]["b"]),
      p["lin"][0]["W"], row(p["lin"][0]["b"]),
      wl2, bl2,
      p["alpha"].reshape(1, 3), p["beta"].reshape(1, 1),
  ]
  out = _tc_combine(m1, m2, x_pad, xg, flat_w)
  return out[:, :1]


# speedup vs baseline: 19.0653x; 19.0653x over previous
import functools

import jax
import jax.numpy as jnp
from jax import lax
from jax.experimental import pallas as pl
from jax.experimental.pallas import tpu as pltpu
from jax.experimental.pallas import tpu_sc as plsc

N_NODES = 10000
N_EDGES = 160000
T = 256
PAIRS = 2 * T
D = 128
HID = 256
N_PAD = 10240
RP_PAD = 10016
COLS_PAD = N_EDGES + 64
W = 64


def _sc_build(nodes, cols, row_ptr, x):
  info = plsc.get_sparse_core_info()
  NC, NS = info.num_cores, info.num_subcores
  NW = NC * NS
  PPW = PAIRS // NW
  mesh = plsc.VectorSubcoreMesh(core_axis_name="c", subcore_axis_name="s")

  @functools.partial(
      pl.kernel, mesh=mesh,
      compiler_params=pltpu.CompilerParams(needs_layout_passes=False),
      out_type=[
          jax.ShapeDtypeStruct((PAIRS, N_PAD), jnp.float32),
          jax.ShapeDtypeStruct((PAIRS, N_PAD), jnp.float32),
          jax.ShapeDtypeStruct((PAIRS, D), jnp.float32),
      ],
      scratch_types=[
          pltpu.VMEM((PPW + 16,), jnp.int32),
          pltpu.VMEM((RP_PAD,), jnp.int32),
          pltpu.VMEM((N_PAD,), jnp.float32),
          pltpu.VMEM((N_PAD,), jnp.float32),
          pltpu.VMEM((W + 16,), jnp.int32),
          pltpu.VMEM((W,), jnp.int32),
          pltpu.VMEM((PPW, D), jnp.float32),
          pltpu.VMEM_SHARED((COLS_PAD,), jnp.int32),
          pltpu.SemaphoreType.DMA,
      ],
  )
  def k(nodes_h, cols_h, rp_h, x_h, m1_h, m2_h, xg_h,
        nodes_v, rp_v, m1row, m2row, c1, c2, xrows, cols_s, sem):
    cid = lax.axis_index("c")
    sid = lax.axis_index("s")
    wid = sid * NC + cid
    base_pair = wid * PPW

    @pl.when(sid == 0)
    def _():
      pltpu.sync_copy(cols_h, cols_s)
    plsc.subcore_barrier()

    pltpu.sync_copy(rp_h, rp_v)
    pltpu.sync_copy(nodes_h.at[pl.ds(base_pair, PPW)],
                    nodes_v.at[pl.ds(0, PPW)])

    idxv = nodes_v[pl.ds(0, 16)]
    pltpu.async_copy(x_h.at[idxv], xrows, sem).wait()
    pltpu.sync_copy(xrows, xg_h.at[pl.ds(base_pair, PPW)])

    lanes = lax.iota(jnp.int32, 16)
    ones = jnp.ones((16,), jnp.float32)

    def sload(ref, i):
      return ref[pl.ds(i, 16)][0]

    def scatter_window(dst_row, chunk, s, b, e):
      for g in range(W // 16):
        col = chunk[pl.ds(g * 16, 16)]
        gl = g * 16 + lanes
        valid = (gl >= (s - b)) & (gl < (e - b))
        plsc.store_scatter(dst_row, [col], ones, mask=valid)

    def fill_one_hop(dst_row, n):
      s = sload(rp_v, n)
      e = sload(rp_v, n + 1)
      b0 = (s // 8) * 8
      nwin = (e - b0 + (W - 1)) // W

      @pl.loop(0, nwin)
      def _(w):
        b = b0 + w * W
        pltpu.sync_copy(cols_s.at[pl.ds(b, W)], c1.at[pl.ds(0, W)])
        scatter_window(dst_row, c1, s, b, e)

    def fill_two_hop(dst_row, n):
      s = sload(rp_v, n)
      e = sload(rp_v, n + 1)
      b0 = (s // 8) * 8
      nwin = (e - b0 + (W - 1)) // W

      @pl.loop(0, nwin)
      def _(w):
        b = b0 + w * W
        pltpu.sync_copy(cols_s.at[pl.ds(b, W)], c1.at[pl.ds(0, W)])
        jlo = jnp.maximum(s - b, 0)
        jhi = jnp.minimum(e - b, W)

        @pl.loop(jlo, jhi)
        def _(j):
          kk = sload(c1, j)
          s2 = sload(rp_v, kk)
          e2 = sload(rp_v, kk + 1)
          b20 = (s2 // 8) * 8
          nw2 = (e2 - b20 + (W - 1)) // W

          @pl.loop(0, nw2)
          def _(w2):
            b2 = b20 + w2 * W
            pltpu.sync_copy(cols_s.at[pl.ds(b2, W)], c2)
            scatter_window(dst_row, c2, s2, b2, e2)

    def pair_body(p, carry):
      n = sload(nodes_v, p)

      def zbody(i, carry2):
        m1row[pl.ds(i * 16, 16)] = jnp.zeros((16,), jnp.float32)
        m2row[pl.ds(i * 16, 16)] = jnp.zeros((16,), jnp.float32)
        return carry2

      lax.fori_loop(0, N_PAD // 16, zbody, 0)
      fill_one_hop(m1row, n)
      fill_two_hop(m2row, n)
      pg = base_pair + p
      pltpu.sync_copy(m1row, m1_h.at[pg])
      pltpu.sync_copy(m2row, m2_h.at[pg])
      return carry

    lax.fori_loop(0, PPW, pair_body, 0)

  return k(nodes, cols, row_ptr, x)


TN = 1024
GSTEPS = N_PAD // TN


def _tc_body(m1_ref, m2_ref, xb_ref, xg_ref,
             w11, b11, w12, b12, w13, b13,
             w21, b21, w22, b22, w23, b23,
             w41, b41, w42, b42, w43, b43,
             wi1, bi1, wi2, bi2,
             wl1, bl1, wl2, bl2,
             alpha, beta,
             out_ref, a1, a2, a3, a4):
  i = pl.program_id(0)

  @pl.when(i == 0)
  def _():
    a1[...] = jnp.zeros_like(a1)
    a2[...] = jnp.zeros_like(a2)
    a3[...] = jnp.zeros_like(a3)
    a4[...] = jnp.zeros_like(a4)

  m1 = m1_ref[...]
  m2 = m2_ref[...]
  xb = xb_ref[...]
  m1s, m1d = m1[:T, :], m1[T:, :]
  m2s, m2d = m2[:T, :], m2[T:, :]
  dot = functools.partial(jnp.dot, preferred_element_type=jnp.float32)
  a1[...] += dot(m1s * m1d, xb)
  a2[...] += dot(m1s * m2d, xb)
  a3[...] += dot(m2s * m1d, xb)
  a4[...] += dot(m2s * m2d, xb)

  @pl.when(i == GSTEPS - 1)
  def _():
    def mlp3(h, wa, ba, wb, bb, wc, bc):
      h = jnp.maximum(dot(h, wa[...]) + ba[...], 0.0)
      h = jnp.maximum(dot(h, wb[...]) + bb[...], 0.0)
      return dot(h, wc[...]) + bc[...]

    xcn1 = mlp3(a1[...], w11, b11, w12, b12, w13, b13)
    xcn2 = mlp3(a2[...], w21, b21, w22, b22, w23, b23)
    xcn3 = mlp3(a3[...], w21, b21, w22, b22, w23, b23)
    xcn4 = mlp3(a4[...], w41, b41, w42, b42, w43, b43)

    xg = xg_ref[...]
    xij_in = xg[:T, :] * xg[T:, :]
    hij = jnp.maximum(dot(xij_in, wi1[...]) + bi1[...], 0.0)
    xij = dot(hij, wi2[...]) + bi2[...]

    sa = jax.nn.sigmoid(alpha[...])
    al0 = sa[0, 0]
    al1 = al0 * sa[0, 1]
    al2 = al1 * sa[0, 2]
    h = al0 * xcn1 + al1 * xcn2 * xcn3 + al2 * xcn4 + beta[0, 0] * xij
    hl = jnp.maximum(dot(h, wl1[...]) + bl1[...], 0.0)
    out_ref[...] = dot(hl, wl2[...]) + bl2[...]


def _tc_combine(m1, m2, x_pad, xg, flat_w):
  full = lambda arr: pl.BlockSpec(arr.shape, lambda i: (0,) * arr.ndim)
  in_specs = [
      pl.BlockSpec((PAIRS, TN), lambda i: (0, i)),
      pl.BlockSpec((PAIRS, TN), lambda i: (0, i)),
      pl.BlockSpec((TN, D), lambda i: (i, 0)),
      full(xg),
  ] + [full(a) for a in flat_w]
  return pl.pallas_call(
      _tc_body,
      grid=(GSTEPS,),
      in_specs=in_specs,
      out_specs=pl.BlockSpec((T, D), lambda i: (0, 0)),
      out_shape=jax.ShapeDtypeStruct((T, D), jnp.float32),
      scratch_shapes=[pltpu.VMEM((T, D), jnp.float32)] * 4,
  )(m1, m2, x_pad, xg, *flat_w)


def kernel(x, adj, tar_ei, params):
  adj0 = adj[0].astype(jnp.int32)
  adj1 = adj[1].astype(jnp.int32)

  order = jnp.argsort(adj0)
  cols_sorted = adj1[order]
  counts = jnp.zeros((N_NODES,), jnp.int32).at[adj0].add(1)
  row_ptr = jnp.concatenate(
      [jnp.zeros((1,), jnp.int32), jnp.cumsum(counts, dtype=jnp.int32)])
  row_ptr = jnp.pad(row_ptr, (0, RP_PAD - (N_NODES + 1)),
                    constant_values=N_EDGES)
  cols_pad = jnp.pad(cols_sorted, (0, COLS_PAD - N_EDGES))
  nodes = jnp.concatenate([tar_ei[0], tar_ei[1]]).astype(jnp.int32)

  m1, m2, xg = _sc_build(nodes, cols_pad, row_ptr, x)

  x_pad = jnp.pad(x, ((0, N_PAD - N_NODES), (0, 0)))

  p = params
  row = lambda b: b.reshape(1, -1)
  wl2 = jnp.pad(p["lin"][1]["W"], ((0, 0), (0, D - 1)))
  bl2 = jnp.pad(row(p["lin"][1]["b"]), ((0, 0), (0, D - 1)))
  flat_w = [
      p["xcn1"][0]["W"], row(p["xcn1"][0]["b"]),
      p["xcn1"][1]["W"], row(p["xcn1"][1]["b"]),
      p["xcn1"][2]["W"], row(p["xcn1"][2]["b"]),
      p["xcn2"][0]["W"], row(p["xcn2"][0]["b"]),
      p["xcn2"][1]["W"], row(p["xcn2"][1]["b"]),
      p["xcn2"][2]["W"], row(p["xcn2"][2]["b"]),
      p["xcn4"][0]["W"], row(p["xcn4"][0]["b"]),
      p["xcn4"][1]["W"], row(p["xcn4"][1]["b"]),
      p["xcn4"][2]["W"], row(p["xcn4"][2]["b"]),
      p["xij"][0]["W"], row(p["xij"][0]["b"]),
      p["xij"][1]["W"], row(p["xij"][1]["b"]),
      p["lin"][0]["W"], row(p["lin"][0]["b"]),
      wl2, bl2,
      p["alpha"].reshape(1, 3), p["beta"].reshape(1, 1),
  ]
  out = _tc_combine(m1, m2, x_pad, xg, flat_w)
  return out[:, :1]


# speedup vs baseline: 19.7104x; 1.0338x over previous
import functools

import jax
import jax.numpy as jnp
from jax import lax
from jax.experimental import pallas as pl
from jax.experimental.pallas import tpu as pltpu
from jax.experimental.pallas import tpu_sc as plsc

N_NODES = 10000
N_EDGES = 160000
T = 256
PAIRS = 2 * T
D = 128
HID = 256
N_PAD = 10240
RP_PAD = 10016
COLS_PAD = N_EDGES + 64
W = 64


def _sc_build(nodes, cols, row_ptr, x):
  info = plsc.get_sparse_core_info()
  NC, NS = info.num_cores, info.num_subcores
  NW = NC * NS
  PPW = PAIRS // NW
  mesh = plsc.VectorSubcoreMesh(core_axis_name="c", subcore_axis_name="s")

  @functools.partial(
      pl.kernel, mesh=mesh,
      compiler_params=pltpu.CompilerParams(needs_layout_passes=False),
      out_type=[
          jax.ShapeDtypeStruct((PAIRS, N_PAD), jnp.float32),
          jax.ShapeDtypeStruct((PAIRS, N_PAD), jnp.float32),
          jax.ShapeDtypeStruct((PAIRS, D), jnp.float32),
      ],
      scratch_types=[
          pltpu.VMEM((PPW + 16,), jnp.int32),
          pltpu.VMEM((RP_PAD,), jnp.int32),
          pltpu.VMEM((N_PAD,), jnp.float32),
          pltpu.VMEM((N_PAD,), jnp.float32),
          pltpu.VMEM((W + 16,), jnp.int32),
          pltpu.VMEM((W,), jnp.int32),
          pltpu.VMEM((PPW, D), jnp.float32),
          pltpu.VMEM_SHARED((COLS_PAD,), jnp.int32),
          pltpu.SemaphoreType.DMA,
      ],
  )
  def k(nodes_h, cols_h, rp_h, x_h, m1_h, m2_h, xg_h,
        nodes_v, rp_v, m1row, m2row, c1, c2, xrows, cols_s, sem):
    cid = lax.axis_index("c")
    sid = lax.axis_index("s")
    wid = sid * NC + cid
    base_pair = wid * PPW

    @pl.when(sid == 0)
    def _():
      pltpu.sync_copy(cols_h, cols_s)
    plsc.subcore_barrier()

    pltpu.sync_copy(rp_h, rp_v)
    pltpu.sync_copy(nodes_h.at[pl.ds(base_pair, PPW)],
                    nodes_v.at[pl.ds(0, PPW)])

    idxv = nodes_v[pl.ds(0, 16)]
    pltpu.async_copy(x_h.at[idxv], xrows, sem).wait()
    pltpu.sync_copy(xrows, xg_h.at[pl.ds(base_pair, PPW)])

    lanes = lax.iota(jnp.int32, 16)
    ones = jnp.ones((16,), jnp.float32)

    def sload(ref, i):
      return ref[pl.ds(i, 16)][0]

    def scatter_window(dst_row, chunk, s, b, e):
      for g in range(W // 16):
        col = chunk[pl.ds(g * 16, 16)]
        gl = g * 16 + lanes
        valid = (gl >= (s - b)) & (gl < (e - b))
        plsc.store_scatter(dst_row, [col], ones, mask=valid)

    def fill_one_hop(dst_row, n):
      s = sload(rp_v, n)
      e = sload(rp_v, n + 1)
      b0 = (s // 8) * 8
      nwin = (e - b0 + (W - 1)) // W

      @pl.loop(0, nwin)
      def _(w):
        b = b0 + w * W
        pltpu.sync_copy(cols_s.at[pl.ds(b, W)], c1.at[pl.ds(0, W)])
        scatter_window(dst_row, c1, s, b, e)

    def fill_two_hop(dst_row, n):
      s = sload(rp_v, n)
      e = sload(rp_v, n + 1)
      b0 = (s // 8) * 8
      nwin = (e - b0 + (W - 1)) // W

      @pl.loop(0, nwin)
      def _(w):
        b = b0 + w * W
        pltpu.sync_copy(cols_s.at[pl.ds(b, W)], c1.at[pl.ds(0, W)])
        jlo = jnp.maximum(s - b, 0)
        jhi = jnp.minimum(e - b, W)

        @pl.loop(jlo, jhi)
        def _(j):
          kk = sload(c1, j)
          s2 = sload(rp_v, kk)
          e2 = sload(rp_v, kk + 1)
          b20 = (s2 // 8) * 8
          nw2 = (e2 - b20 + (W - 1)) // W

          @pl.loop(0, nw2)
          def _(w2):
            b2 = b20 + w2 * W
            pltpu.sync_copy(cols_s.at[pl.ds(b2, W)], c2)
            scatter_window(dst_row, c2, s2, b2, e2)

    def pair_body(p, carry):
      n = sload(nodes_v, p)

      def zbody(i, carry2):
        m1row[pl.ds(i * 16, 16)] = jnp.zeros((16,), jnp.float32)
        m2row[pl.ds(i * 16, 16)] = jnp.zeros((16,), jnp.float32)
        return carry2

      lax.fori_loop(0, N_PAD // 16, zbody, 0)
      fill_one_hop(m1row, n)
      fill_two_hop(m2row, n)
      pg = base_pair + p
      pltpu.sync_copy(m1row, m1_h.at[pg])
      pltpu.sync_copy(m2row, m2_h.at[pg])
      return carry

    lax.fori_loop(0, PPW, pair_body, 0)

  return k(nodes, cols, row_ptr, x)


TN = 1024
GSTEPS = N_PAD // TN


def _tc_body(m1_ref, m2_ref, xb_ref, xg_ref,
             w11, b11, w12, b12, w13, b13,
             w21, b21, w22, b22, w23, b23,
             w41, b41, w42, b42, w43, b43,
             wi1, bi1, wi2, bi2,
             wl1, bl1, wl2, bl2,
             alpha, beta,
             out_ref, a1, a2, a3, a4):
  i = pl.program_id(0)

  @pl.when(i == 0)
  def _():
    a1[...] = jnp.zeros_like(a1)
    a2[...] = jnp.zeros_like(a2)
    a3[...] = jnp.zeros_like(a3)
    a4[...] = jnp.zeros_like(a4)

  m1 = m1_ref[...]
  m2 = m2_ref[...]
  xb = xb_ref[...]
  m1s, m1d = m1[:T, :], m1[T:, :]
  m2s, m2d = m2[:T, :], m2[T:, :]
  dot = functools.partial(jnp.dot, preferred_element_type=jnp.float32)
  a1[...] += dot(m1s * m1d, xb)
  a2[...] += dot(m1s * m2d, xb)
  a3[...] += dot(m2s * m1d, xb)
  a4[...] += dot(m2s * m2d, xb)

  @pl.when(i == GSTEPS - 1)
  def _():
    def mlp3(h, wa, ba, wb, bb, wc, bc):
      h = jnp.maximum(dot(h, wa[...]) + ba[...], 0.0)
      h = jnp.maximum(dot(h, wb[...]) + bb[...], 0.0)
      return dot(h, wc[...]) + bc[...]

    xcn1 = mlp3(a1[...], w11, b11, w12, b12, w13, b13)
    xcn2 = mlp3(a2[...], w21, b21, w22, b22, w23, b23)
    xcn3 = mlp3(a3[...], w21, b21, w22, b22, w23, b23)
    xcn4 = mlp3(a4[...], w41, b41, w42, b42, w43, b43)

    xg = xg_ref[...]
    xij_in = xg[:T, :] * xg[T:, :]
    hij = jnp.maximum(dot(xij_in, wi1[...]) + bi1[...], 0.0)
    xij = dot(hij, wi2[...]) + bi2[...]

    sa = jax.nn.sigmoid(alpha[...])
    al0 = sa[0, 0]
    al1 = al0 * sa[0, 1]
    al2 = al1 * sa[0, 2]
    h = al0 * xcn1 + al1 * xcn2 * xcn3 + al2 * xcn4 + beta[0, 0] * xij
    hl = jnp.maximum(dot(h, wl1[...]) + bl1[...], 0.0)
    out_ref[...] = dot(hl, wl2[...]) + bl2[...]


def _tc_combine(m1, m2, x_pad, xg, flat_w):
  full = lambda arr: pl.BlockSpec(arr.shape, lambda i: (0,) * arr.ndim)
  in_specs = [
      pl.BlockSpec((PAIRS, TN), lambda i: (0, i)),
      pl.BlockSpec((PAIRS, TN), lambda i: (0, i)),
      pl.BlockSpec((TN, D), lambda i: (i, 0)),
      full(xg),
  ] + [full(a) for a in flat_w]
  return pl.pallas_call(
      _tc_body,
      grid=(GSTEPS,),
      in_specs=in_specs,
      out_specs=pl.BlockSpec((T, D), lambda i: (0, 0)),
      out_shape=jax.ShapeDtypeStruct((T, D), jnp.float32),
      scratch_shapes=[pltpu.VMEM((T, D), jnp.float32)] * 4,
  )(m1, m2, x_pad, xg, *flat_w)


def kernel(x, adj, tar_ei, params):
  adj0 = adj[0].astype(jnp.int32)
  adj1 = adj[1].astype(jnp.int32)

  packed = adj0 * 16384 + adj1
  cols_sorted = jnp.sort(packed) & 16383
  counts = jnp.zeros((N_NODES,), jnp.int32).at[adj0].add(1)
  row_ptr = jnp.concatenate(
      [jnp.zeros((1,), jnp.int32), jnp.cumsum(counts, dtype=jnp.int32)])
  row_ptr = jnp.pad(row_ptr, (0, RP_PAD - (N_NODES + 1)),
                    constant_values=N_EDGES)
  cols_pad = jnp.pad(cols_sorted, (0, COLS_PAD - N_EDGES))
  nodes = jnp.concatenate([tar_ei[0], tar_ei[1]]).astype(jnp.int32)

  m1, m2, xg = _sc_build(nodes, cols_pad, row_ptr, x)

  x_pad = jnp.pad(x, ((0, N_PAD - N_NODES), (0, 0)))

  p = params
  row = lambda b: b.reshape(1, -1)
  wl2 = jnp.pad(p["lin"][1]["W"], ((0, 0), (0, D - 1)))
  bl2 = jnp.pad(row(p["lin"][1]["b"]), ((0, 0), (0, D - 1)))
  flat_w = [
      p["xcn1"][0]["W"], row(p["xcn1"][0]["b"]),
      p["xcn1"][1]["W"], row(p["xcn1"][1]["b"]),
      p["xcn1"][2]["W"], row(p["xcn1"][2]["b"]),
      p["xcn2"][0]["W"], row(p["xcn2"][0]["b"]),
      p["xcn2"][1]["W"], row(p["xcn2"][1]["b"]),
      p["xcn2"][2]["W"], row(p["xcn2"][2]["b"]),
      p["xcn4"][0]["W"], row(p["xcn4"][0]["b"]),
      p["xcn4"][1]["W"], row(p["xcn4"][1]["b"]),
      p["xcn4"][2]["W"], row(p["xcn4"][2]["b"]),
      p["xij"][0]["W"], row(p["xij"][0]["b"]),
      p["xij"][1]["W"], row(p["xij"][1]["b"]),
      p["lin"][0]["W"], row(p["lin"][0]["b"]),
      wl2, bl2,
      p["alpha"].reshape(1, 3), p["beta"].reshape(1, 1),
  ]
  out = _tc_combine(m1, m2, x_pad, xg, flat_w)
  return out[:, :1]
